# baseline (speedup 1.0000x reference)
import functools

import jax
import jax.numpy as jnp
import numpy as np
from jax import lax
from jax.experimental import pallas as pl
from jax.experimental.pallas import tpu as pltpu
from jax.experimental.pallas import tpu_sc as plsc

_N = 8192
_G = 128
_D = 1024
_L = 1024
_K = 128
_BN = 512

_NC = 2
_NS = 16
_NW = _NC * _NS
_SEG_PER_W = _G // _NW
_LANES = 16

_NEG = np.float32(-3.4e38)

_U1_HEX = (
    "187c713e28e2693e3c6ca13e68a9b83ef8a4e33ec28e7a3fd0dc533ee892fa3ea022a73e"
    "f0071e3fda5f673fdcf60f3f64a21d3f56bf5d3fc49c173fcc72dc3ec070143e18d5ca3e"
    "8483be3eee66513f1c17d43e5c6ba23e54d1c73ef6451b3f0089603d6050783d50451e3e"
    "523e153f9250603fc44c0f3f2688423fc85fe03ea8e4983e20c81e3e24b4323f6a5a713f"
    "5a61433fdcd2643f128d393f3e79213fb85ca93e9c00a13ee0d1643e8af32e3f8ec2173f"
    "84c05b3fcea63f3f1e5d463f00bfc03b5c54203f742dee3e0a233f3f0064443f629b073f"
    "c039073d3a715a3f30df763e2cd1653f40da3d3ea0f01f3f0c0de13e009c9a3df47b153f"
    "64e8d13ef849763f6cdbf23e6c5a173fda31073f5e07793f005e623c1096e83d12fb263f"
    "f035923ecc524c3f182ebe3e724f5e3f007a163ebef7113f201d5c3d8c14483f8692373f"
    "902d533ecc89863ed42e963eec6d973e68c9d23e3886e93ed053333fee4a4c3fa051413e"
    "c4a1b63e1638753fa0aab73db83aae3ea8740f3ec0e9b23e02f36b3ffc09453f3c49683f"
    "4c6e603f24dabb3ef47c893e20a8e43d6c5d6f3f0ca3ce3efeaa323fa4a2a23e00a5cf3c"
    "9a06333f30f01c3f00da0c3cc8f92b3f6230263f46b1423f2094af3d4ce6123fee5a4d3f"
    "e8219b3e6c49a33efa03033fc824b63e00a4e03ceee9363f36e95a3f009b563e405ba53e"
    "a0bd023da8604a3e"
)
_U2_HEX = (
    "b43dd53e206f183e38396b3e3c46453fc00e723f544e2d3f847c0d3f9817d73e78de183f"
    "5817693f00f5fb3eccf1073f98b20c3ebe02033f80e7393dc0dce03e8221223f1ad0373f"
    "1016743fde4e743f6452093f88752b3ec46c953ef8e3a43ee638643f26154e3f5cd5ec3e"
    "522e293f4e31683fa6b61f3fb870ee3ee4e0f23e80d5413d526e3a3ff0f6e23e5814b63e"
    "24f99b3e6412c13e3254233fa8371f3ebc954d3f4caa9d3eb0a6483e6c38eb3ecc88fd3e"
    "c87b583f665a413f545c7c3fd65a223fb088593e2c9f063ff65e6e3f2063243d440d993e"
    "54249c3ea8a0bf3edc7f3b3f5c1b883e308a573f40fbca3c882a643fe8e7fe3eac7bf23e"
    "12905f3f4880343e0e2d4c3fc0f9363ec0b1353e04cf583f001b433d585e493f001f3e3d"
    "d036cb3e9819ea3ec0277f3e74aaf53e5cde9b3ea039cc3d8e274a3f9cb4903ea294263f"
    "9817503fc0d2d63c3837413fae40533fe8ba0d3f8038223c04096e3fc86aca3e082b353e"
    "de226d3f9040573e40a6433e54bda03e20a8773f60adef3e7808393e500c133e88c0753f"
    "3806053ee099243d0e4c133fe4e0f83eb88a243ffe8a6f3fdaba413f20ab5f3d98bf233e"
    "c095f13cd85c963e8436ef3e58dacc3e00ed833c9a9c1a3f7e8f773f2064623f0042763c"
    "322e303f188e0b3ed0545f3fa4dbcc3ec0c89b3c388b7b3e8457a13eb8c1413e6675403f"
    "b0dbcc3e501f323e"
)
_U1 = np.frombuffer(bytes.fromhex(_U1_HEX), dtype=np.float32).copy()
_U2 = np.frombuffer(bytes.fromhex(_U2_HEX), dtype=np.float32).copy()


def _mlp_body(x_ref, w1_ref, b1_ref, w2_ref, b2_ref, w3_ref, b3_ref,
              eabs_ref, rm_ref, es_ref):
    h = jnp.dot(x_ref[...], w1_ref[...], preferred_element_type=jnp.float32)
    h = jnp.maximum(h + b1_ref[...], 0.0)
    h = jnp.dot(h, w2_ref[...], preferred_element_type=jnp.float32)
    h = jnp.maximum(h + b2_ref[...], 0.0)
    l = jnp.dot(h, w3_ref[...], preferred_element_type=jnp.float32) + b3_ref[...]
    eabs = jnp.exp(l)
    eabs_ref[...] = eabs
    rm_ref[...] = jnp.max(l, axis=1).reshape(1, 1, _BN)
    es_ref[...] = jnp.sum(eabs, axis=1).reshape(1, 1, _BN)


def _mlp_masses(x, W1, b1, W2, b2, W3, b3):
    nblk = _N // _BN
    out_shapes = (
        jax.ShapeDtypeStruct((_N, _K), jnp.float32),
        jax.ShapeDtypeStruct((nblk, 1, _BN), jnp.float32),
        jax.ShapeDtypeStruct((nblk, 1, _BN), jnp.float32),
    )
    return pl.pallas_call(
        _mlp_body,
        grid=(nblk,),
        in_specs=[
            pl.BlockSpec((_BN, _D), lambda i: (i, 0)),
            pl.BlockSpec((_D, _L), lambda i: (0, 0)),
            pl.BlockSpec((1, _L), lambda i: (0, 0)),
            pl.BlockSpec((_L, _L), lambda i: (0, 0)),
            pl.BlockSpec((1, _L), lambda i: (0, 0)),
            pl.BlockSpec((_L, _K), lambda i: (0, 0)),
            pl.BlockSpec((1, _K), lambda i: (0, 0)),
        ],
        out_specs=(
            pl.BlockSpec((_BN, _K), lambda i: (i, 0)),
            pl.BlockSpec((1, 1, _BN), lambda i: (i, 0, 0)),
            pl.BlockSpec((1, 1, _BN), lambda i: (i, 0, 0)),
        ),
        out_shape=out_shapes,
    )(x, W1, b1.reshape(1, _L), W2, b2.reshape(1, _L), W3, b3.reshape(1, _K))


def _sc_body(rm_hbm, es_hbm, sid_hbm, stop_hbm, u1_hbm, u2_hbm, eabs_hbm,
             u_out, stopp_out, node_out, spec_out,
             sid_v, rm_v, es_v, cs_v, stop_v, u1_v, u2_v, rows_v,
             resf_v, resi_v, idx_v, sem):
    wid = lax.axis_index("s") * _NC + lax.axis_index("c")
    iota = lax.iota(jnp.int32, _LANES)

    def sload(ref, idx):
        return ref[pl.ds(idx, _LANES)][0]

    pltpu.sync_copy(sid_hbm, sid_v.at[pl.ds(0, _N)])
    pltpu.sync_copy(rm_hbm, rm_v.at[pl.ds(0, _N)])
    pltpu.sync_copy(es_hbm, es_v.at[pl.ds(0, _N)])
    pltpu.sync_copy(stop_hbm, stop_v.at[pl.ds(0, _G)])
    pltpu.sync_copy(u1_hbm, u1_v.at[pl.ds(0, _G)])
    pltpu.sync_copy(u2_hbm, u2_v.at[pl.ds(0, _G)])
    rm_v[pl.ds(_N, _LANES)] = jnp.zeros((_LANES,), jnp.float32)
    es_v[pl.ds(_N, _LANES)] = jnp.zeros((_LANES,), jnp.float32)

    g0 = wid * _SEG_PER_W

    def lower_bound(g):
        pos = jnp.int32(0)
        b = _N // 2
        while b >= 1:
            v = sload(sid_v, pos + (b - 1))
            pos = jnp.where(v < g, pos + b, pos)
            b //= 2
        return jnp.where(g >= jnp.int32(_G), jnp.int32(_N), pos)

    starts = [lower_bound(g0 + jj) for jj in range(_SEG_PER_W + 1)]

    res_u = jnp.zeros((_LANES,), jnp.float32)
    res_stop = jnp.zeros((_LANES,), jnp.float32)
    res_node = jnp.zeros((_LANES,), jnp.int32)
    nodes = []
    handles = []

    for j in range(_SEG_PER_W):
        g = g0 + j
        s = starts[j]
        e = starts[j + 1]
        nch = lax.div(e - s + (_LANES - 1), _LANES)
        stop_g = sload(stop_v, g)
        u1_g = sload(u1_v, g)

        def body_a(k, m, s=s, e=e):
            off = s + k * _LANES
            vals = rm_v[pl.ds(off, _LANES)]
            mask = (off + iota) < e
            return jnp.maximum(m, jnp.where(mask, vals, _NEG))

        mvec = pl.loop(0, nch,
                       init_carry=jnp.full((_LANES,), _NEG, jnp.float32))(body_a)
        M = jnp.maximum(jnp.max(mvec), stop_g)

        def body_b(k, carry, s=s, e=e):
            off = s + k * _LANES
            ve = es_v[pl.ds(off, _LANES)]
            mask = (off + iota) < e
            cum = plsc.cumsum(jnp.where(mask, ve, 0.0)) + carry
            cs_v[pl.ds(off, _LANES)] = cum
            return cum[_LANES - 1]

        t_node = pl.loop(0, nch, init_carry=jnp.float32(0.0))(body_b)

        expnegm = jnp.exp(jnp.full((_LANES,), -M, jnp.float32))
        expstop = jnp.exp(jnp.full((_LANES,), stop_g - M, jnp.float32))
        norm_vec = t_node * expnegm + expstop
        u_vec = expnegm / norm_vec
        stop_vec = expstop / norm_vec

        r1 = t_node * (jnp.float32(1.0) - u1_g)
        o = jnp.int32(0)
        span = e - s
        b = _N // 2
        while b >= 1:
            cand = o + b
            pos = jnp.minimum(s + cand - 1, jnp.int32(_N - 1))
            v = sload(cs_v, pos)
            ok = jnp.logical_and(cand <= span, v < r1)
            o = jnp.where(ok, cand, o)
            b //= 2
        node = s + jnp.minimum(o, span - 1)
        nodes.append(node)

        handles.append(pltpu.async_copy(eabs_hbm.at[node], rows_v.at[j], sem))

        res_u = jnp.where(iota == j, u_vec, res_u)
        res_stop = jnp.where(iota == j, stop_vec, res_stop)
        res_node = jnp.where(iota == j, node, res_node)

    res_spec = jnp.zeros((_LANES,), jnp.int32)
    for j in range(_SEG_PER_W):
        handles[j].wait()
        u2_g = sload(u2_v, g0 + j)
        carry = jnp.float32(0.0)
        cums = []
        for kk in range(_K // _LANES):
            wv = rows_v[j, pl.ds(kk * _LANES, _LANES)]
            cum = plsc.cumsum(wv) + carry
            cums.append(cum)
            carry = cum[_LANES - 1]
        r2 = carry * (jnp.float32(1.0) - u2_g)
        sp = jnp.int32(0)
        for kk in range(_K // _LANES):
            lt = (cums[kk] < r2).astype(jnp.int32)
            sp = sp + plsc.cumsum(lt)[_LANES - 1]
        sp = jnp.minimum(sp, jnp.int32(_K - 1))
        res_spec = jnp.where(iota == j, sp, res_spec)

    idx_v[...] = g0 + jnp.where(iota < _SEG_PER_W, iota, 0)
    resf_v[...] = res_u
    resi_v[...] = res_node
    pltpu.async_copy(resf_v.at[pl.ds(0, _SEG_PER_W)],
                     u_out.at[idx_v.at[pl.ds(0, _SEG_PER_W)]], sem).wait()
    resf_v[...] = res_stop
    pltpu.async_copy(resf_v.at[pl.ds(0, _SEG_PER_W)],
                     stopp_out.at[idx_v.at[pl.ds(0, _SEG_PER_W)]], sem).wait()
    pltpu.async_copy(resi_v.at[pl.ds(0, _SEG_PER_W)],
                     node_out.at[idx_v.at[pl.ds(0, _SEG_PER_W)]], sem).wait()
    resi_v[...] = res_spec
    pltpu.async_copy(resi_v.at[pl.ds(0, _SEG_PER_W)],
                     spec_out.at[idx_v.at[pl.ds(0, _SEG_PER_W)]], sem).wait()


def _sc_segment_sample(rm, es, sid, stop, u1, u2, eabs):
    mesh = plsc.VectorSubcoreMesh(core_axis_name="c", subcore_axis_name="s")
    fn = pl.kernel(
        _sc_body,
        out_type=[
            jax.ShapeDtypeStruct((_G,), jnp.float32),
            jax.ShapeDtypeStruct((_G,), jnp.float32),
            jax.ShapeDtypeStruct((_G,), jnp.int32),
            jax.ShapeDtypeStruct((_G,), jnp.int32),
        ],
        mesh=mesh,
        compiler_params=pltpu.CompilerParams(needs_layout_passes=False),
        scratch_types=[
            pltpu.VMEM((_N + _LANES,), jnp.int32),
            pltpu.VMEM((_N + _LANES,), jnp.float32),
            pltpu.VMEM((_N + _LANES,), jnp.float32),
            pltpu.VMEM((_N + _LANES,), jnp.float32),
            pltpu.VMEM((_G + _LANES,), jnp.float32),
            pltpu.VMEM((_G + _LANES,), jnp.float32),
            pltpu.VMEM((_G + _LANES,), jnp.float32),
            pltpu.VMEM((_SEG_PER_W, _K), jnp.float32),
            pltpu.VMEM((_LANES,), jnp.float32),
            pltpu.VMEM((_LANES,), jnp.int32),
            pltpu.VMEM((_LANES,), jnp.int32),
            pltpu.SemaphoreType.DMA,
        ],
    )
    return fn(rm, es, sid, stop, u1, u2, eabs)


def _probs_body(e_ref, sid_ref, u_ref, out_ref):
    sid = sid_ref[...]
    g = lax.broadcasted_iota(jnp.int32, (1, _G), 1)
    onehot = (sid == g).astype(jnp.float32)
    ucol = jnp.transpose(u_ref[...], (1, 0))
    t = jnp.dot(onehot, ucol, preferred_element_type=jnp.float32)
    out_ref[...] = e_ref[...] * t


def _probs(eabs, sid2d, U):
    return pl.pallas_call(
        _probs_body,
        grid=(_N // _BN,),
        in_specs=[
            pl.BlockSpec((_BN, _K), lambda i: (i, 0)),
            pl.BlockSpec((_BN, 1), lambda i: (i, 0)),
            pl.BlockSpec((1, _G), lambda i: (0, 0)),
        ],
        out_specs=pl.BlockSpec((_BN, _K), lambda i: (i, 0)),
        out_shape=jax.ShapeDtypeStruct((_N, _K), jnp.float32),
    )(eabs, sid2d, U.reshape(1, _G))


def kernel(node_embeddings, stop_logits, segment_ids, W1, b1, W2, b2, W3, b3):
    eabs, rm3, es3 = _mlp_masses(node_embeddings, W1, b1, W2, b2, W3, b3)
    U, stop_probs, node_indices, species_indices = _sc_segment_sample(
        rm3.reshape(_N), es3.reshape(_N), segment_ids, stop_logits,
        jnp.asarray(_U1), jnp.asarray(_U2), eabs)
    species_probs = _probs(eabs, segment_ids.reshape(_N, 1), U)
    return species_probs, stop_probs, node_indices, species_indices


# speedup vs baseline: 1.0810x; 1.0810x over previous
import functools

import jax
import jax.numpy as jnp
import numpy as np
from jax import lax
from jax.experimental import pallas as pl
from jax.experimental.pallas import tpu as pltpu
from jax.experimental.pallas import tpu_sc as plsc

_N = 8192
_G = 128
_D = 1024
_L = 1024
_K = 128
_BN = 512

_NC = 2
_NS = 16
_NW = _NC * _NS
_SEG_PER_W = _G // _NW
_LANES = 16

_NEG = np.float32(-3.4e38)

_U1_HEX = (
    "187c713e28e2693e3c6ca13e68a9b83ef8a4e33ec28e7a3fd0dc533ee892fa3ea022a73e"
    "f0071e3fda5f673fdcf60f3f64a21d3f56bf5d3fc49c173fcc72dc3ec070143e18d5ca3e"
    "8483be3eee66513f1c17d43e5c6ba23e54d1c73ef6451b3f0089603d6050783d50451e3e"
    "523e153f9250603fc44c0f3f2688423fc85fe03ea8e4983e20c81e3e24b4323f6a5a713f"
    "5a61433fdcd2643f128d393f3e79213fb85ca93e9c00a13ee0d1643e8af32e3f8ec2173f"
    "84c05b3fcea63f3f1e5d463f00bfc03b5c54203f742dee3e0a233f3f0064443f629b073f"
    "c039073d3a715a3f30df763e2cd1653f40da3d3ea0f01f3f0c0de13e009c9a3df47b153f"
    "64e8d13ef849763f6cdbf23e6c5a173fda31073f5e07793f005e623c1096e83d12fb263f"
    "f035923ecc524c3f182ebe3e724f5e3f007a163ebef7113f201d5c3d8c14483f8692373f"
    "902d533ecc89863ed42e963eec6d973e68c9d23e3886e93ed053333fee4a4c3fa051413e"
    "c4a1b63e1638753fa0aab73db83aae3ea8740f3ec0e9b23e02f36b3ffc09453f3c49683f"
    "4c6e603f24dabb3ef47c893e20a8e43d6c5d6f3f0ca3ce3efeaa323fa4a2a23e00a5cf3c"
    "9a06333f30f01c3f00da0c3cc8f92b3f6230263f46b1423f2094af3d4ce6123fee5a4d3f"
    "e8219b3e6c49a33efa03033fc824b63e00a4e03ceee9363f36e95a3f009b563e405ba53e"
    "a0bd023da8604a3e"
)
_U2_HEX = (
    "b43dd53e206f183e38396b3e3c46453fc00e723f544e2d3f847c0d3f9817d73e78de183f"
    "5817693f00f5fb3eccf1073f98b20c3ebe02033f80e7393dc0dce03e8221223f1ad0373f"
    "1016743fde4e743f6452093f88752b3ec46c953ef8e3a43ee638643f26154e3f5cd5ec3e"
    "522e293f4e31683fa6b61f3fb870ee3ee4e0f23e80d5413d526e3a3ff0f6e23e5814b63e"
    "24f99b3e6412c13e3254233fa8371f3ebc954d3f4caa9d3eb0a6483e6c38eb3ecc88fd3e"
    "c87b583f665a413f545c7c3fd65a223fb088593e2c9f063ff65e6e3f2063243d440d993e"
    "54249c3ea8a0bf3edc7f3b3f5c1b883e308a573f40fbca3c882a643fe8e7fe3eac7bf23e"
    "12905f3f4880343e0e2d4c3fc0f9363ec0b1353e04cf583f001b433d585e493f001f3e3d"
    "d036cb3e9819ea3ec0277f3e74aaf53e5cde9b3ea039cc3d8e274a3f9cb4903ea294263f"
    "9817503fc0d2d63c3837413fae40533fe8ba0d3f8038223c04096e3fc86aca3e082b353e"
    "de226d3f9040573e40a6433e54bda03e20a8773f60adef3e7808393e500c133e88c0753f"
    "3806053ee099243d0e4c133fe4e0f83eb88a243ffe8a6f3fdaba413f20ab5f3d98bf233e"
    "c095f13cd85c963e8436ef3e58dacc3e00ed833c9a9c1a3f7e8f773f2064623f0042763c"
    "322e303f188e0b3ed0545f3fa4dbcc3ec0c89b3c388b7b3e8457a13eb8c1413e6675403f"
    "b0dbcc3e501f323e"
)
_U1 = np.frombuffer(bytes.fromhex(_U1_HEX), dtype=np.float32).copy()
_U2 = np.frombuffer(bytes.fromhex(_U2_HEX), dtype=np.float32).copy()


def _mlp_body(x_ref, w1_ref, b1_ref, w2_ref, b2_ref, w3_ref, b3_ref,
              eabs_ref, es_ref):
    h = jnp.dot(x_ref[...], w1_ref[...], preferred_element_type=jnp.float32)
    h = jnp.maximum(h + b1_ref[...], 0.0)
    h = jnp.dot(h, w2_ref[...], preferred_element_type=jnp.float32)
    h = jnp.maximum(h + b2_ref[...], 0.0)
    l = jnp.dot(h, w3_ref[...], preferred_element_type=jnp.float32) + b3_ref[...]
    eabs = jnp.exp(l)
    eabs_ref[...] = eabs
    es_ref[...] = jnp.sum(eabs, axis=1, keepdims=True)


def _mlp_masses(x, W1, b1, W2, b2, W3, b3):
    nblk = _N // _BN
    out_shapes = (
        jax.ShapeDtypeStruct((_N, _K), jnp.float32),
        jax.ShapeDtypeStruct((_N, 1), jnp.float32),
    )
    return pl.pallas_call(
        _mlp_body,
        grid=(nblk,),
        in_specs=[
            pl.BlockSpec((_BN, _D), lambda i: (i, 0)),
            pl.BlockSpec((_D, _L), lambda i: (0, 0)),
            pl.BlockSpec((1, _L), lambda i: (0, 0)),
            pl.BlockSpec((_L, _L), lambda i: (0, 0)),
            pl.BlockSpec((1, _L), lambda i: (0, 0)),
            pl.BlockSpec((_L, _K), lambda i: (0, 0)),
            pl.BlockSpec((1, _K), lambda i: (0, 0)),
        ],
        out_specs=(
            pl.BlockSpec((_BN, _K), lambda i: (i, 0)),
            pl.BlockSpec((_BN, 1), lambda i: (i, 0)),
        ),
        out_shape=out_shapes,
    )(x, W1, b1.reshape(1, _L), W2, b2.reshape(1, _L), W3, b3.reshape(1, _K))


def _sc_body(es_hbm, sid_hbm, stop_hbm, u1_hbm, u2_hbm, eabs_hbm,
             f_out, i_out,
             sid_v, es_v, cs_v, stop_v, u1_v, u2_v, rows_v, spc_v, tmp_v,
             resf_v, resi_v, idx_v, sem):
    wid = lax.axis_index("s") * _NC + lax.axis_index("c")
    iota = lax.iota(jnp.int32, _LANES)

    def splat(ref, g):
        return plsc.load_gather(ref, [jnp.full((_LANES,), g, jnp.int32)])

    pltpu.sync_copy(sid_hbm, sid_v.at[pl.ds(0, _N)])
    pltpu.sync_copy(es_hbm, es_v.at[pl.ds(0, _N)])
    pltpu.sync_copy(stop_hbm, stop_v)
    pltpu.sync_copy(u1_hbm, u1_v)
    pltpu.sync_copy(u2_hbm, u2_v)
    es_v[pl.ds(_N, _LANES)] = jnp.zeros((_LANES,), jnp.float32)

    g0 = wid * _SEG_PER_W

    gl = g0 + jnp.minimum(iota, _SEG_PER_W)
    pos = jnp.zeros((_LANES,), jnp.int32)
    b = _N // 2
    while b >= 1:
        probe = pos + (b - 1)
        v = plsc.load_gather(sid_v, [probe])
        pos = jnp.where(v < gl, pos + b, pos)
        b //= 2
    pos = jnp.where(gl >= jnp.int32(_G), jnp.int32(_N), pos)
    starts = [pos[jj] for jj in range(_SEG_PER_W + 1)]

    res_f = jnp.zeros((_LANES,), jnp.float32)
    r1v = jnp.zeros((_LANES,), jnp.float32)

    for j in range(_SEG_PER_W):
        s = starts[j]
        e = starts[j + 1]
        nch = lax.div(e - s + (_LANES - 1), _LANES)

        def body_b(k, carryv, s=s, e=e):
            off = s + k * _LANES
            ve = es_v[pl.ds(off, _LANES)]
            mask = (off + iota) < e
            cum = plsc.cumsum(jnp.where(mask, ve, 0.0)) + carryv
            cs_v[pl.ds(off, _LANES)] = cum
            return splat(cs_v, off + (_LANES - 1))

        tvec = pl.loop(0, nch,
                       init_carry=jnp.zeros((_LANES,), jnp.float32))(body_b)

        expstop = jnp.exp(splat(stop_v, g0 + j))
        norm_vec = tvec + expstop
        res_f = jnp.where(iota == j, jnp.float32(1.0) / norm_vec, res_f)
        res_f = jnp.where(iota == (_SEG_PER_W + j), expstop / norm_vec, res_f)
        r1j = tvec * (jnp.float32(1.0) - splat(u1_v, g0 + j))
        r1v = jnp.where(iota == j, r1j, r1v)

    s_vec = pos
    e_vec = jnp.zeros((_LANES,), jnp.int32)
    for j in range(_SEG_PER_W):
        e_vec = jnp.where(iota == j, starts[j + 1], e_vec)
    span = e_vec - s_vec
    o = jnp.zeros((_LANES,), jnp.int32)
    b = _N // 2
    while b >= 1:
        cand = o + b
        probe = jnp.minimum(s_vec + cand - 1, jnp.int32(_N - 1))
        v = plsc.load_gather(cs_v, [probe])
        ok = jnp.logical_and(cand <= span, v < r1v)
        o = jnp.where(ok, cand, o)
        b //= 2
    node_vec = s_vec + jnp.minimum(o, span - 1)
    res_i = jnp.where(iota < _SEG_PER_W, node_vec, 0)

    handles = [
        pltpu.async_copy(eabs_hbm.at[node_vec[j]], rows_v.at[j], sem)
        for j in range(_SEG_PER_W)
    ]

    for j in range(_SEG_PER_W):
        handles[j].wait()
        carryv = jnp.zeros((_LANES,), jnp.float32)
        for kk in range(_K // _LANES):
            cum = plsc.cumsum(rows_v[j, pl.ds(kk * _LANES, _LANES)]) + carryv
            spc_v[pl.ds(kk * _LANES, _LANES)] = cum
            carryv = splat(spc_v, kk * _LANES + (_LANES - 1))
        r2v = carryv * (jnp.float32(1.0) - splat(u2_v, g0 + j))
        cnt = jnp.zeros((_LANES,), jnp.int32)
        for kk in range(_K // _LANES):
            cnt = cnt + (spc_v[pl.ds(kk * _LANES, _LANES)] < r2v).astype(jnp.int32)
        tmp_v[...] = plsc.cumsum(cnt)
        spv = jnp.minimum(splat(tmp_v, _LANES - 1), jnp.int32(_K - 1))
        res_i = jnp.where(iota == (_SEG_PER_W + j), spv, res_i)

    idx_v[...] = jnp.where(iota < _SEG_PER_W, g0 + iota,
                           _G + g0 + (iota - _SEG_PER_W))
    resf_v[...] = res_f
    resi_v[...] = res_i
    h1 = pltpu.async_copy(resf_v.at[pl.ds(0, 2 * _SEG_PER_W)],
                          f_out.at[idx_v.at[pl.ds(0, 2 * _SEG_PER_W)]], sem)
    h2 = pltpu.async_copy(resi_v.at[pl.ds(0, 2 * _SEG_PER_W)],
                          i_out.at[idx_v.at[pl.ds(0, 2 * _SEG_PER_W)]], sem)
    h1.wait()
    h2.wait()


def _sc_segment_sample(es, sid, stop, u1, u2, eabs):
    mesh = plsc.VectorSubcoreMesh(core_axis_name="c", subcore_axis_name="s")
    fn = pl.kernel(
        _sc_body,
        out_type=[
            jax.ShapeDtypeStruct((2 * _G,), jnp.float32),
            jax.ShapeDtypeStruct((2 * _G,), jnp.int32),
        ],
        mesh=mesh,
        compiler_params=pltpu.CompilerParams(needs_layout_passes=False),
        scratch_types=[
            pltpu.VMEM((_N + _LANES,), jnp.int32),
            pltpu.VMEM((_N + _LANES,), jnp.float32),
            pltpu.VMEM((_N + _LANES,), jnp.float32),
            pltpu.VMEM((_G,), jnp.float32),
            pltpu.VMEM((_G,), jnp.float32),
            pltpu.VMEM((_G,), jnp.float32),
            pltpu.VMEM((_SEG_PER_W, _K), jnp.float32),
            pltpu.VMEM((_K,), jnp.float32),
            pltpu.VMEM((_LANES,), jnp.int32),
            pltpu.VMEM((_LANES,), jnp.float32),
            pltpu.VMEM((_LANES,), jnp.int32),
            pltpu.VMEM((_LANES,), jnp.int32),
            pltpu.SemaphoreType.DMA,
        ],
    )
    return fn(es, sid, stop, u1, u2, eabs)


def _probs_body(e_ref, sid_ref, u_ref, out_ref):
    sid = sid_ref[...]
    g = lax.broadcasted_iota(jnp.int32, (1, _G), 1)
    onehot = (sid == g).astype(jnp.float32)
    ucol = jnp.transpose(u_ref[...], (1, 0))
    t = jnp.dot(onehot, ucol, preferred_element_type=jnp.float32)
    out_ref[...] = e_ref[...] * t


def _probs(eabs, sid2d, U):
    return pl.pallas_call(
        _probs_body,
        grid=(_N // _BN,),
        in_specs=[
            pl.BlockSpec((_BN, _K), lambda i: (i, 0)),
            pl.BlockSpec((_BN, 1), lambda i: (i, 0)),
            pl.BlockSpec((1, _G), lambda i: (0, 0)),
        ],
        out_specs=pl.BlockSpec((_BN, _K), lambda i: (i, 0)),
        out_shape=jax.ShapeDtypeStruct((_N, _K), jnp.float32),
    )(eabs, sid2d, U.reshape(1, _G))


def kernel(node_embeddings, stop_logits, segment_ids, W1, b1, W2, b2, W3, b3):
    eabs, es2 = _mlp_masses(node_embeddings, W1, b1, W2, b2, W3, b3)
    f_out, i_out = _sc_segment_sample(
        es2.reshape(_N), segment_ids, stop_logits,
        jnp.asarray(_U1), jnp.asarray(_U2), eabs)
    U = f_out[:_G]
    stop_probs = f_out[_G:]
    node_indices = i_out[:_G]
    species_indices = i_out[_G:]
    species_probs = _probs(eabs, segment_ids.reshape(_N, 1), U)
    return species_probs, stop_probs, node_indices, species_indices


# speedup vs baseline: 1.4216x; 1.3151x over previous
import functools

import jax
import jax.numpy as jnp
import numpy as np
from jax import lax
from jax.experimental import pallas as pl
from jax.experimental.pallas import tpu as pltpu
from jax.experimental.pallas import tpu_sc as plsc

_N = 8192
_G = 128
_D = 1024
_L = 1024
_K = 128
_BN = 512

_NC = 2
_NS = 16
_NW = _NC * _NS
_SEG_PER_W = _G // _NW
_LANES = 16

_NEG = np.float32(-3.4e38)

_U1_HEX = (
    "187c713e28e2693e3c6ca13e68a9b83ef8a4e33ec28e7a3fd0dc533ee892fa3ea022a73e"
    "f0071e3fda5f673fdcf60f3f64a21d3f56bf5d3fc49c173fcc72dc3ec070143e18d5ca3e"
    "8483be3eee66513f1c17d43e5c6ba23e54d1c73ef6451b3f0089603d6050783d50451e3e"
    "523e153f9250603fc44c0f3f2688423fc85fe03ea8e4983e20c81e3e24b4323f6a5a713f"
    "5a61433fdcd2643f128d393f3e79213fb85ca93e9c00a13ee0d1643e8af32e3f8ec2173f"
    "84c05b3fcea63f3f1e5d463f00bfc03b5c54203f742dee3e0a233f3f0064443f629b073f"
    "c039073d3a715a3f30df763e2cd1653f40da3d3ea0f01f3f0c0de13e009c9a3df47b153f"
    "64e8d13ef849763f6cdbf23e6c5a173fda31073f5e07793f005e623c1096e83d12fb263f"
    "f035923ecc524c3f182ebe3e724f5e3f007a163ebef7113f201d5c3d8c14483f8692373f"
    "902d533ecc89863ed42e963eec6d973e68c9d23e3886e93ed053333fee4a4c3fa051413e"
    "c4a1b63e1638753fa0aab73db83aae3ea8740f3ec0e9b23e02f36b3ffc09453f3c49683f"
    "4c6e603f24dabb3ef47c893e20a8e43d6c5d6f3f0ca3ce3efeaa323fa4a2a23e00a5cf3c"
    "9a06333f30f01c3f00da0c3cc8f92b3f6230263f46b1423f2094af3d4ce6123fee5a4d3f"
    "e8219b3e6c49a33efa03033fc824b63e00a4e03ceee9363f36e95a3f009b563e405ba53e"
    "a0bd023da8604a3e"
)
_U2_HEX = (
    "b43dd53e206f183e38396b3e3c46453fc00e723f544e2d3f847c0d3f9817d73e78de183f"
    "5817693f00f5fb3eccf1073f98b20c3ebe02033f80e7393dc0dce03e8221223f1ad0373f"
    "1016743fde4e743f6452093f88752b3ec46c953ef8e3a43ee638643f26154e3f5cd5ec3e"
    "522e293f4e31683fa6b61f3fb870ee3ee4e0f23e80d5413d526e3a3ff0f6e23e5814b63e"
    "24f99b3e6412c13e3254233fa8371f3ebc954d3f4caa9d3eb0a6483e6c38eb3ecc88fd3e"
    "c87b583f665a413f545c7c3fd65a223fb088593e2c9f063ff65e6e3f2063243d440d993e"
    "54249c3ea8a0bf3edc7f3b3f5c1b883e308a573f40fbca3c882a643fe8e7fe3eac7bf23e"
    "12905f3f4880343e0e2d4c3fc0f9363ec0b1353e04cf583f001b433d585e493f001f3e3d"
    "d036cb3e9819ea3ec0277f3e74aaf53e5cde9b3ea039cc3d8e274a3f9cb4903ea294263f"
    "9817503fc0d2d63c3837413fae40533fe8ba0d3f8038223c04096e3fc86aca3e082b353e"
    "de226d3f9040573e40a6433e54bda03e20a8773f60adef3e7808393e500c133e88c0753f"
    "3806053ee099243d0e4c133fe4e0f83eb88a243ffe8a6f3fdaba413f20ab5f3d98bf233e"
    "c095f13cd85c963e8436ef3e58dacc3e00ed833c9a9c1a3f7e8f773f2064623f0042763c"
    "322e303f188e0b3ed0545f3fa4dbcc3ec0c89b3c388b7b3e8457a13eb8c1413e6675403f"
    "b0dbcc3e501f323e"
)
_U1 = np.frombuffer(bytes.fromhex(_U1_HEX), dtype=np.float32).copy()
_U2 = np.frombuffer(bytes.fromhex(_U2_HEX), dtype=np.float32).copy()


def _mlp_body(x_ref, w1_ref, b1_ref, w2_ref, b2_ref, w3_ref, b3_ref,
              eabs_ref, es_ref):
    h = jnp.dot(x_ref[...], w1_ref[...], preferred_element_type=jnp.float32)
    h = jnp.maximum(h + b1_ref[...], 0.0)
    h = jnp.dot(h, w2_ref[...], preferred_element_type=jnp.float32)
    h = jnp.maximum(h + b2_ref[...], 0.0)
    l = jnp.dot(h, w3_ref[...], preferred_element_type=jnp.float32) + b3_ref[...]
    eabs = jnp.exp(l)
    eabs_ref[...] = eabs
    es_ref[...] = jnp.sum(eabs, axis=1, keepdims=True)


def _mlp_masses(x, W1, b1, W2, b2, W3, b3):
    nblk = _N // _BN
    out_shapes = (
        jax.ShapeDtypeStruct((_N, _K), jnp.float32),
        jax.ShapeDtypeStruct((_N, 1), jnp.float32),
    )
    return pl.pallas_call(
        _mlp_body,
        grid=(nblk,),
        in_specs=[
            pl.BlockSpec((_BN, _D), lambda i: (i, 0)),
            pl.BlockSpec((_D, _L), lambda i: (0, 0)),
            pl.BlockSpec((1, _L), lambda i: (0, 0)),
            pl.BlockSpec((_L, _L), lambda i: (0, 0)),
            pl.BlockSpec((1, _L), lambda i: (0, 0)),
            pl.BlockSpec((_L, _K), lambda i: (0, 0)),
            pl.BlockSpec((1, _K), lambda i: (0, 0)),
        ],
        out_specs=(
            pl.BlockSpec((_BN, _K), lambda i: (i, 0)),
            pl.BlockSpec((_BN, 1), lambda i: (i, 0)),
        ),
        out_shape=out_shapes,
    )(x, W1, b1.reshape(1, _L), W2, b2.reshape(1, _L), W3, b3.reshape(1, _K))


_SEG16 = _G // _NS


def _sc_body(es_hbm, sid_hbm, stop_hbm, u1_hbm, u2_hbm, eabs_hbm,
             u_out, stop_out, node_out, spec_out,
             sid_v, es_v, cs_v, stop_v, u1_v, u2_v, rows_v, spc_v, tmp_v,
             st_u, st_stop, st_node, st_spec, sem, sem2):
    cidx = lax.axis_index("c")
    wid = lax.axis_index("s")
    iota = lax.iota(jnp.int32, _LANES)

    def splat(ref, g):
        return plsc.load_gather(ref, [jnp.full((_LANES,), g, jnp.int32)])

    h_in = [
        pltpu.async_copy(sid_hbm, sid_v.at[pl.ds(0, _N)], sem2),
        pltpu.async_copy(es_hbm, es_v.at[pl.ds(0, _N)], sem2),
        pltpu.async_copy(stop_hbm, stop_v, sem2),
        pltpu.async_copy(u1_hbm, u1_v, sem2),
        pltpu.async_copy(u2_hbm, u2_v, sem2),
    ]
    for h in h_in:
        h.wait()
    es_v[pl.ds(_N, _LANES)] = jnp.zeros((_LANES,), jnp.float32)

    g0 = wid * _SEG16

    gl = g0 + jnp.minimum(iota, _SEG16)
    pos = jnp.zeros((_LANES,), jnp.int32)
    b = _N // 2
    while b >= 1:
        probe = pos + (b - 1)
        v = plsc.load_gather(sid_v, [probe])
        pos = jnp.where(v < gl, pos + b, pos)
        b //= 2
    pos = jnp.where(gl >= jnp.int32(_G), jnp.int32(_N), pos)
    starts = [pos[jj] for jj in range(_SEG16 + 1)]

    res_u = jnp.zeros((_LANES,), jnp.float32)
    res_stop = jnp.zeros((_LANES,), jnp.float32)
    r1v = jnp.zeros((_LANES,), jnp.float32)

    for j in range(_SEG16):
        s = starts[j]
        e = starts[j + 1]
        nch = lax.div(e - s + (_LANES - 1), _LANES)

        def body_b(k, carryv, s=s, e=e):
            off = s + k * _LANES
            ve = es_v[pl.ds(off, _LANES)]
            mask = (off + iota) < e
            cum = plsc.cumsum(jnp.where(mask, ve, 0.0)) + carryv
            cs_v[pl.ds(off, _LANES)] = cum
            return splat(cs_v, off + (_LANES - 1))

        tvec = pl.loop(0, nch,
                       init_carry=jnp.zeros((_LANES,), jnp.float32))(body_b)

        expstop = jnp.exp(splat(stop_v, g0 + j))
        norm_vec = tvec + expstop
        res_u = jnp.where(iota == j, jnp.float32(1.0) / norm_vec, res_u)
        res_stop = jnp.where(iota == j, expstop / norm_vec, res_stop)
        r1j = tvec * (jnp.float32(1.0) - splat(u1_v, g0 + j))
        r1v = jnp.where(iota == j, r1j, r1v)

    s_vec = pos
    e_vec = jnp.zeros((_LANES,), jnp.int32)
    for j in range(_SEG16):
        e_vec = jnp.where(iota == j, starts[j + 1], e_vec)
    span = e_vec - s_vec
    o = jnp.zeros((_LANES,), jnp.int32)
    b = _N // 2
    while b >= 1:
        cand = o + b
        probe = jnp.minimum(s_vec + cand - 1, jnp.int32(_N - 1))
        v = plsc.load_gather(cs_v, [probe])
        ok = jnp.logical_and(cand <= span, v < r1v)
        o = jnp.where(ok, cand, o)
        b //= 2
    node_vec = s_vec + jnp.minimum(o, span - 1)
    res_node = jnp.where(iota < _SEG16, node_vec, 0)

    handles = [
        pltpu.async_copy(eabs_hbm.at[node_vec[j]], rows_v.at[j], sem)
        for j in range(_SEG16)
    ]
    for h in handles:
        h.wait()

    res_spec = jnp.zeros((_LANES,), jnp.int32)
    for j in range(_SEG16):
        carryv = jnp.zeros((_LANES,), jnp.float32)
        for kk in range(_K // _LANES):
            cum = plsc.cumsum(rows_v[j, pl.ds(kk * _LANES, _LANES)]) + carryv
            spc_v[pl.ds(kk * _LANES, _LANES)] = cum
            carryv = splat(spc_v, kk * _LANES + (_LANES - 1))
        r2v = carryv * (jnp.float32(1.0) - splat(u2_v, g0 + j))
        cnt = jnp.zeros((_LANES,), jnp.int32)
        for kk in range(_K // _LANES):
            cnt = cnt + (spc_v[pl.ds(kk * _LANES, _LANES)] < r2v).astype(jnp.int32)
        tmp_v[...] = plsc.cumsum(cnt)
        spv = jnp.minimum(splat(tmp_v, _LANES - 1), jnp.int32(_K - 1))
        res_spec = jnp.where(iota == j, spv, res_spec)

    @pl.when(cidx == 0)
    def _publish():
        st_u[...] = res_u
        st_stop[...] = res_stop
        st_node[...] = res_node
        st_spec[...] = res_spec
        hs = [
            pltpu.async_copy(st_u.at[pl.ds(0, _SEG16)],
                             u_out.at[pl.ds(g0, _SEG16)], sem),
            pltpu.async_copy(st_stop.at[pl.ds(0, _SEG16)],
                             stop_out.at[pl.ds(g0, _SEG16)], sem),
            pltpu.async_copy(st_node.at[pl.ds(0, _SEG16)],
                             node_out.at[pl.ds(g0, _SEG16)], sem),
            pltpu.async_copy(st_spec.at[pl.ds(0, _SEG16)],
                             spec_out.at[pl.ds(g0, _SEG16)], sem),
        ]
        for h in hs:
            h.wait()


def _sc_segment_sample(es, sid, stop, u1, u2, eabs):
    mesh = plsc.VectorSubcoreMesh(core_axis_name="c", subcore_axis_name="s")
    fn = pl.kernel(
        _sc_body,
        out_type=[
            jax.ShapeDtypeStruct((_G,), jnp.float32),
            jax.ShapeDtypeStruct((_G,), jnp.float32),
            jax.ShapeDtypeStruct((_G,), jnp.int32),
            jax.ShapeDtypeStruct((_G,), jnp.int32),
        ],
        mesh=mesh,
        compiler_params=pltpu.CompilerParams(needs_layout_passes=False),
        scratch_types=[
            pltpu.VMEM((_N + _LANES,), jnp.int32),
            pltpu.VMEM((_N + _LANES,), jnp.float32),
            pltpu.VMEM((_N + _LANES,), jnp.float32),
            pltpu.VMEM((_G,), jnp.float32),
            pltpu.VMEM((_G,), jnp.float32),
            pltpu.VMEM((_G,), jnp.float32),
            pltpu.VMEM((_SEG16, _K), jnp.float32),
            pltpu.VMEM((_K,), jnp.float32),
            pltpu.VMEM((_LANES,), jnp.int32),
            pltpu.VMEM((_LANES,), jnp.float32),
            pltpu.VMEM((_LANES,), jnp.float32),
            pltpu.VMEM((_LANES,), jnp.int32),
            pltpu.VMEM((_LANES,), jnp.int32),
            pltpu.SemaphoreType.DMA,
            pltpu.SemaphoreType.DMA,
        ],
    )
    return fn(es, sid, stop, u1, u2, eabs)


def _probs_body(e_ref, sid_ref, u_ref, out_ref):
    sid = sid_ref[...]
    g = lax.broadcasted_iota(jnp.int32, (1, _G), 1)
    onehot = (sid == g).astype(jnp.float32)
    ucol = jnp.transpose(u_ref[...], (1, 0))
    t = jnp.dot(onehot, ucol, preferred_element_type=jnp.float32)
    out_ref[...] = e_ref[...] * t


_BN2 = 2048


def _probs(eabs, sid2d, U):
    return pl.pallas_call(
        _probs_body,
        grid=(_N // _BN2,),
        in_specs=[
            pl.BlockSpec((_BN2, _K), lambda i: (i, 0)),
            pl.BlockSpec((_BN2, 1), lambda i: (i, 0)),
            pl.BlockSpec((1, _G), lambda i: (0, 0)),
        ],
        out_specs=pl.BlockSpec((_BN2, _K), lambda i: (i, 0)),
        out_shape=jax.ShapeDtypeStruct((_N, _K), jnp.float32),
    )(eabs, sid2d, U.reshape(1, _G))


def kernel(node_embeddings, stop_logits, segment_ids, W1, b1, W2, b2, W3, b3):
    eabs, es2 = _mlp_masses(node_embeddings, W1, b1, W2, b2, W3, b3)
    U, stop_probs, node_indices, species_indices = _sc_segment_sample(
        es2.reshape(_N), segment_ids, stop_logits,
        jnp.asarray(_U1), jnp.asarray(_U2), eabs)
    species_probs = _probs(eabs, segment_ids.reshape(_N, 1), U)
    return species_probs, stop_probs, node_indices, species_indices


# speedup vs baseline: 1.4984x; 1.0540x over previous
import functools

import jax
import jax.numpy as jnp
import numpy as np
from jax import lax
from jax.experimental import pallas as pl
from jax.experimental.pallas import tpu as pltpu
from jax.experimental.pallas import tpu_sc as plsc

_N = 8192
_G = 128
_D = 1024
_L = 1024
_K = 128
_BN = 1024

_NC = 2
_NS = 16
_NW = _NC * _NS
_SEG_PER_W = _G // _NW
_LANES = 16

_NEG = np.float32(-3.4e38)

_U1_HEX = (
    "187c713e28e2693e3c6ca13e68a9b83ef8a4e33ec28e7a3fd0dc533ee892fa3ea022a73e"
    "f0071e3fda5f673fdcf60f3f64a21d3f56bf5d3fc49c173fcc72dc3ec070143e18d5ca3e"
    "8483be3eee66513f1c17d43e5c6ba23e54d1c73ef6451b3f0089603d6050783d50451e3e"
    "523e153f9250603fc44c0f3f2688423fc85fe03ea8e4983e20c81e3e24b4323f6a5a713f"
    "5a61433fdcd2643f128d393f3e79213fb85ca93e9c00a13ee0d1643e8af32e3f8ec2173f"
    "84c05b3fcea63f3f1e5d463f00bfc03b5c54203f742dee3e0a233f3f0064443f629b073f"
    "c039073d3a715a3f30df763e2cd1653f40da3d3ea0f01f3f0c0de13e009c9a3df47b153f"
    "64e8d13ef849763f6cdbf23e6c5a173fda31073f5e07793f005e623c1096e83d12fb263f"
    "f035923ecc524c3f182ebe3e724f5e3f007a163ebef7113f201d5c3d8c14483f8692373f"
    "902d533ecc89863ed42e963eec6d973e68c9d23e3886e93ed053333fee4a4c3fa051413e"
    "c4a1b63e1638753fa0aab73db83aae3ea8740f3ec0e9b23e02f36b3ffc09453f3c49683f"
    "4c6e603f24dabb3ef47c893e20a8e43d6c5d6f3f0ca3ce3efeaa323fa4a2a23e00a5cf3c"
    "9a06333f30f01c3f00da0c3cc8f92b3f6230263f46b1423f2094af3d4ce6123fee5a4d3f"
    "e8219b3e6c49a33efa03033fc824b63e00a4e03ceee9363f36e95a3f009b563e405ba53e"
    "a0bd023da8604a3e"
)
_U2_HEX = (
    "b43dd53e206f183e38396b3e3c46453fc00e723f544e2d3f847c0d3f9817d73e78de183f"
    "5817693f00f5fb3eccf1073f98b20c3ebe02033f80e7393dc0dce03e8221223f1ad0373f"
    "1016743fde4e743f6452093f88752b3ec46c953ef8e3a43ee638643f26154e3f5cd5ec3e"
    "522e293f4e31683fa6b61f3fb870ee3ee4e0f23e80d5413d526e3a3ff0f6e23e5814b63e"
    "24f99b3e6412c13e3254233fa8371f3ebc954d3f4caa9d3eb0a6483e6c38eb3ecc88fd3e"
    "c87b583f665a413f545c7c3fd65a223fb088593e2c9f063ff65e6e3f2063243d440d993e"
    "54249c3ea8a0bf3edc7f3b3f5c1b883e308a573f40fbca3c882a643fe8e7fe3eac7bf23e"
    "12905f3f4880343e0e2d4c3fc0f9363ec0b1353e04cf583f001b433d585e493f001f3e3d"
    "d036cb3e9819ea3ec0277f3e74aaf53e5cde9b3ea039cc3d8e274a3f9cb4903ea294263f"
    "9817503fc0d2d63c3837413fae40533fe8ba0d3f8038223c04096e3fc86aca3e082b353e"
    "de226d3f9040573e40a6433e54bda03e20a8773f60adef3e7808393e500c133e88c0753f"
    "3806053ee099243d0e4c133fe4e0f83eb88a243ffe8a6f3fdaba413f20ab5f3d98bf233e"
    "c095f13cd85c963e8436ef3e58dacc3e00ed833c9a9c1a3f7e8f773f2064623f0042763c"
    "322e303f188e0b3ed0545f3fa4dbcc3ec0c89b3c388b7b3e8457a13eb8c1413e6675403f"
    "b0dbcc3e501f323e"
)
_U1 = np.frombuffer(bytes.fromhex(_U1_HEX), dtype=np.float32).copy()
_U2 = np.frombuffer(bytes.fromhex(_U2_HEX), dtype=np.float32).copy()


def _mlp_body(x_ref, w1_ref, b1_ref, w2_ref, b2_ref, w3_ref, b3_ref,
              eabs_ref, es_ref):
    h = jnp.dot(x_ref[...], w1_ref[...], preferred_element_type=jnp.float32)
    h = jnp.maximum(h + b1_ref[...], 0.0)
    h = jnp.dot(h, w2_ref[...], preferred_element_type=jnp.float32)
    h = jnp.maximum(h + b2_ref[...], 0.0)
    l = jnp.dot(h, w3_ref[...], preferred_element_type=jnp.float32) + b3_ref[...]
    eabs = jnp.exp(l)
    eabs_ref[...] = eabs
    es_ref[...] = jnp.sum(eabs, axis=1, keepdims=True)


def _mlp_masses(x, W1, b1, W2, b2, W3, b3):
    nblk = _N // _BN
    out_shapes = (
        jax.ShapeDtypeStruct((_N, _K), jnp.float32),
        jax.ShapeDtypeStruct((_N, 1), jnp.float32),
    )
    return pl.pallas_call(
        _mlp_body,
        grid=(nblk,),
        in_specs=[
            pl.BlockSpec((_BN, _D), lambda i: (i, 0)),
            pl.BlockSpec((_D, _L), lambda i: (0, 0)),
            pl.BlockSpec((1, _L), lambda i: (0, 0)),
            pl.BlockSpec((_L, _L), lambda i: (0, 0)),
            pl.BlockSpec((1, _L), lambda i: (0, 0)),
            pl.BlockSpec((_L, _K), lambda i: (0, 0)),
            pl.BlockSpec((1, _K), lambda i: (0, 0)),
        ],
        out_specs=(
            pl.BlockSpec((_BN, _K), lambda i: (i, 0)),
            pl.BlockSpec((_BN, 1), lambda i: (i, 0)),
        ),
        out_shape=out_shapes,
    )(x, W1, b1.reshape(1, _L), W2, b2.reshape(1, _L), W3, b3.reshape(1, _K))


_SEG16 = _G // _NS


def _sc_body(es_hbm, sid_hbm, stop_hbm, u1_hbm, u2_hbm, eabs_hbm,
             u_out, stop_out, node_out, spec_out,
             sid_v, es_v, cs_v, stop_v, u1_v, u2_v, rows_v, spc_v, tmp_v,
             st_u, st_stop, st_node, st_spec, sem, sem2):
    cidx = lax.axis_index("c")
    wid = lax.axis_index("s")
    iota = lax.iota(jnp.int32, _LANES)

    def splat(ref, g):
        return plsc.load_gather(ref, [jnp.full((_LANES,), g, jnp.int32)])

    h_in = [
        pltpu.async_copy(sid_hbm, sid_v.at[pl.ds(0, _N)], sem2),
        pltpu.async_copy(es_hbm, es_v.at[pl.ds(0, _N)], sem2),
        pltpu.async_copy(stop_hbm, stop_v, sem2),
        pltpu.async_copy(u1_hbm, u1_v, sem2),
        pltpu.async_copy(u2_hbm, u2_v, sem2),
    ]
    for h in h_in:
        h.wait()
    es_v[pl.ds(_N, _LANES)] = jnp.zeros((_LANES,), jnp.float32)

    g0 = wid * _SEG16

    gl = g0 + jnp.minimum(iota, _SEG16)
    pos = jnp.zeros((_LANES,), jnp.int32)
    b = _N // 2
    while b >= 1:
        probe = pos + (b - 1)
        v = plsc.load_gather(sid_v, [probe])
        pos = jnp.where(v < gl, pos + b, pos)
        b //= 2
    pos = jnp.where(gl >= jnp.int32(_G), jnp.int32(_N), pos)
    starts = [pos[jj] for jj in range(_SEG16 + 1)]

    res_u = jnp.zeros((_LANES,), jnp.float32)
    res_stop = jnp.zeros((_LANES,), jnp.float32)
    r1v = jnp.zeros((_LANES,), jnp.float32)

    for j in range(_SEG16):
        s = starts[j]
        e = starts[j + 1]
        nch = lax.div(e - s + (_LANES - 1), _LANES)

        def body_b(k, carryv, s=s, e=e):
            off = s + k * _LANES
            ve = es_v[pl.ds(off, _LANES)]
            mask = (off + iota) < e
            cum = plsc.cumsum(jnp.where(mask, ve, 0.0)) + carryv
            cs_v[pl.ds(off, _LANES)] = cum
            return splat(cs_v, off + (_LANES - 1))

        tvec = pl.loop(0, nch,
                       init_carry=jnp.zeros((_LANES,), jnp.float32))(body_b)

        expstop = jnp.exp(splat(stop_v, g0 + j))
        norm_vec = tvec + expstop
        res_u = jnp.where(iota == j, jnp.float32(1.0) / norm_vec, res_u)
        res_stop = jnp.where(iota == j, expstop / norm_vec, res_stop)
        r1j = tvec * (jnp.float32(1.0) - splat(u1_v, g0 + j))
        r1v = jnp.where(iota == j, r1j, r1v)

    s_vec = pos
    e_vec = jnp.zeros((_LANES,), jnp.int32)
    for j in range(_SEG16):
        e_vec = jnp.where(iota == j, starts[j + 1], e_vec)
    span = e_vec - s_vec
    o = jnp.zeros((_LANES,), jnp.int32)
    b = _N // 2
    while b >= 1:
        cand = o + b
        probe = jnp.minimum(s_vec + cand - 1, jnp.int32(_N - 1))
        v = plsc.load_gather(cs_v, [probe])
        ok = jnp.logical_and(cand <= span, v < r1v)
        o = jnp.where(ok, cand, o)
        b //= 2
    node_vec = s_vec + jnp.minimum(o, span - 1)
    res_node = jnp.where(iota < _SEG16, node_vec, 0)

    handles = [
        pltpu.async_copy(eabs_hbm.at[node_vec[j]], rows_v.at[j], sem)
        for j in range(_SEG16)
    ]
    for h in handles:
        h.wait()

    res_spec = jnp.zeros((_LANES,), jnp.int32)
    for j in range(_SEG16):
        carryv = jnp.zeros((_LANES,), jnp.float32)
        for kk in range(_K // _LANES):
            cum = plsc.cumsum(rows_v[j, pl.ds(kk * _LANES, _LANES)]) + carryv
            spc_v[pl.ds(kk * _LANES, _LANES)] = cum
            carryv = splat(spc_v, kk * _LANES + (_LANES - 1))
        r2v = carryv * (jnp.float32(1.0) - splat(u2_v, g0 + j))
        cnt = jnp.zeros((_LANES,), jnp.int32)
        for kk in range(_K // _LANES):
            cnt = cnt + (spc_v[pl.ds(kk * _LANES, _LANES)] < r2v).astype(jnp.int32)
        tmp_v[...] = plsc.cumsum(cnt)
        spv = jnp.minimum(splat(tmp_v, _LANES - 1), jnp.int32(_K - 1))
        res_spec = jnp.where(iota == j, spv, res_spec)

    @pl.when(cidx == 0)
    def _publish():
        st_u[...] = res_u
        st_stop[...] = res_stop
        st_node[...] = res_node
        st_spec[...] = res_spec
        hs = [
            pltpu.async_copy(st_u.at[pl.ds(0, _SEG16)],
                             u_out.at[pl.ds(g0, _SEG16)], sem),
            pltpu.async_copy(st_stop.at[pl.ds(0, _SEG16)],
                             stop_out.at[pl.ds(g0, _SEG16)], sem),
            pltpu.async_copy(st_node.at[pl.ds(0, _SEG16)],
                             node_out.at[pl.ds(g0, _SEG16)], sem),
            pltpu.async_copy(st_spec.at[pl.ds(0, _SEG16)],
                             spec_out.at[pl.ds(g0, _SEG16)], sem),
        ]
        for h in hs:
            h.wait()


def _sc_segment_sample(es, sid, stop, u1, u2, eabs):
    mesh = plsc.VectorSubcoreMesh(core_axis_name="c", subcore_axis_name="s")
    fn = pl.kernel(
        _sc_body,
        out_type=[
            jax.ShapeDtypeStruct((_G,), jnp.float32),
            jax.ShapeDtypeStruct((_G,), jnp.float32),
            jax.ShapeDtypeStruct((_G,), jnp.int32),
            jax.ShapeDtypeStruct((_G,), jnp.int32),
        ],
        mesh=mesh,
        compiler_params=pltpu.CompilerParams(needs_layout_passes=False),
        scratch_types=[
            pltpu.VMEM((_N + _LANES,), jnp.int32),
            pltpu.VMEM((_N + _LANES,), jnp.float32),
            pltpu.VMEM((_N + _LANES,), jnp.float32),
            pltpu.VMEM((_G,), jnp.float32),
            pltpu.VMEM((_G,), jnp.float32),
            pltpu.VMEM((_G,), jnp.float32),
            pltpu.VMEM((_SEG16, _K), jnp.float32),
            pltpu.VMEM((_K,), jnp.float32),
            pltpu.VMEM((_LANES,), jnp.int32),
            pltpu.VMEM((_LANES,), jnp.float32),
            pltpu.VMEM((_LANES,), jnp.float32),
            pltpu.VMEM((_LANES,), jnp.int32),
            pltpu.VMEM((_LANES,), jnp.int32),
            pltpu.SemaphoreType.DMA,
            pltpu.SemaphoreType.DMA,
        ],
    )
    return fn(es, sid, stop, u1, u2, eabs)


def _probs_body(e_ref, sid_ref, u_ref, out_ref):
    sid = sid_ref[...]
    g = lax.broadcasted_iota(jnp.int32, (1, _G), 1)
    onehot = (sid == g).astype(jnp.float32)
    ucol = jnp.transpose(u_ref[...], (1, 0))
    t = jnp.dot(onehot, ucol, preferred_element_type=jnp.float32)
    out_ref[...] = e_ref[...] * t


_BN2 = 4096


def _probs(eabs, sid2d, U):
    return pl.pallas_call(
        _probs_body,
        grid=(_N // _BN2,),
        in_specs=[
            pl.BlockSpec((_BN2, _K), lambda i: (i, 0)),
            pl.BlockSpec((_BN2, 1), lambda i: (i, 0)),
            pl.BlockSpec((1, _G), lambda i: (0, 0)),
        ],
        out_specs=pl.BlockSpec((_BN2, _K), lambda i: (i, 0)),
        out_shape=jax.ShapeDtypeStruct((_N, _K), jnp.float32),
    )(eabs, sid2d, U.reshape(1, _G))


def kernel(node_embeddings, stop_logits, segment_ids, W1, b1, W2, b2, W3, b3):
    eabs, es2 = _mlp_masses(node_embeddings, W1, b1, W2, b2, W3, b3)
    U, stop_probs, node_indices, species_indices = _sc_segment_sample(
        es2.reshape(_N), segment_ids, stop_logits,
        jnp.asarray(_U1), jnp.asarray(_U2), eabs)
    species_probs = _probs(eabs, segment_ids.reshape(_N, 1), U)
    return species_probs, stop_probs, node_indices, species_indices


# speedup vs baseline: 1.5071x; 1.0058x over previous
import functools

import jax
import jax.numpy as jnp
import numpy as np
from jax import lax
from jax.experimental import pallas as pl
from jax.experimental.pallas import tpu as pltpu
from jax.experimental.pallas import tpu_sc as plsc

_N = 8192
_G = 128
_D = 1024
_L = 1024
_K = 128
_BN = 2048

_NC = 2
_NS = 16
_NW = _NC * _NS
_SEG_PER_W = _G // _NW
_LANES = 16

_NEG = np.float32(-3.4e38)

_U1_HEX = (
    "187c713e28e2693e3c6ca13e68a9b83ef8a4e33ec28e7a3fd0dc533ee892fa3ea022a73e"
    "f0071e3fda5f673fdcf60f3f64a21d3f56bf5d3fc49c173fcc72dc3ec070143e18d5ca3e"
    "8483be3eee66513f1c17d43e5c6ba23e54d1c73ef6451b3f0089603d6050783d50451e3e"
    "523e153f9250603fc44c0f3f2688423fc85fe03ea8e4983e20c81e3e24b4323f6a5a713f"
    "5a61433fdcd2643f128d393f3e79213fb85ca93e9c00a13ee0d1643e8af32e3f8ec2173f"
    "84c05b3fcea63f3f1e5d463f00bfc03b5c54203f742dee3e0a233f3f0064443f629b073f"
    "c039073d3a715a3f30df763e2cd1653f40da3d3ea0f01f3f0c0de13e009c9a3df47b153f"
    "64e8d13ef849763f6cdbf23e6c5a173fda31073f5e07793f005e623c1096e83d12fb263f"
    "f035923ecc524c3f182ebe3e724f5e3f007a163ebef7113f201d5c3d8c14483f8692373f"
    "902d533ecc89863ed42e963eec6d973e68c9d23e3886e93ed053333fee4a4c3fa051413e"
    "c4a1b63e1638753fa0aab73db83aae3ea8740f3ec0e9b23e02f36b3ffc09453f3c49683f"
    "4c6e603f24dabb3ef47c893e20a8e43d6c5d6f3f0ca3ce3efeaa323fa4a2a23e00a5cf3c"
    "9a06333f30f01c3f00da0c3cc8f92b3f6230263f46b1423f2094af3d4ce6123fee5a4d3f"
    "e8219b3e6c49a33efa03033fc824b63e00a4e03ceee9363f36e95a3f009b563e405ba53e"
    "a0bd023da8604a3e"
)
_U2_HEX = (
    "b43dd53e206f183e38396b3e3c46453fc00e723f544e2d3f847c0d3f9817d73e78de183f"
    "5817693f00f5fb3eccf1073f98b20c3ebe02033f80e7393dc0dce03e8221223f1ad0373f"
    "1016743fde4e743f6452093f88752b3ec46c953ef8e3a43ee638643f26154e3f5cd5ec3e"
    "522e293f4e31683fa6b61f3fb870ee3ee4e0f23e80d5413d526e3a3ff0f6e23e5814b63e"
    "24f99b3e6412c13e3254233fa8371f3ebc954d3f4caa9d3eb0a6483e6c38eb3ecc88fd3e"
    "c87b583f665a413f545c7c3fd65a223fb088593e2c9f063ff65e6e3f2063243d440d993e"
    "54249c3ea8a0bf3edc7f3b3f5c1b883e308a573f40fbca3c882a643fe8e7fe3eac7bf23e"
    "12905f3f4880343e0e2d4c3fc0f9363ec0b1353e04cf583f001b433d585e493f001f3e3d"
    "d036cb3e9819ea3ec0277f3e74aaf53e5cde9b3ea039cc3d8e274a3f9cb4903ea294263f"
    "9817503fc0d2d63c3837413fae40533fe8ba0d3f8038223c04096e3fc86aca3e082b353e"
    "de226d3f9040573e40a6433e54bda03e20a8773f60adef3e7808393e500c133e88c0753f"
    "3806053ee099243d0e4c133fe4e0f83eb88a243ffe8a6f3fdaba413f20ab5f3d98bf233e"
    "c095f13cd85c963e8436ef3e58dacc3e00ed833c9a9c1a3f7e8f773f2064623f0042763c"
    "322e303f188e0b3ed0545f3fa4dbcc3ec0c89b3c388b7b3e8457a13eb8c1413e6675403f"
    "b0dbcc3e501f323e"
)
_U1 = np.frombuffer(bytes.fromhex(_U1_HEX), dtype=np.float32).copy()
_U2 = np.frombuffer(bytes.fromhex(_U2_HEX), dtype=np.float32).copy()


def _mlp_body(x_ref, w1_ref, b1_ref, w2_ref, b2_ref, w3_ref, b3_ref,
              eabs_ref, es_ref):
    h = jnp.dot(x_ref[...], w1_ref[...], preferred_element_type=jnp.float32)
    h = jnp.maximum(h + b1_ref[...], 0.0)
    h = jnp.dot(h, w2_ref[...], preferred_element_type=jnp.float32)
    h = jnp.maximum(h + b2_ref[...], 0.0)
    l = jnp.dot(h, w3_ref[...], preferred_element_type=jnp.float32) + b3_ref[...]
    eabs = jnp.exp(l)
    eabs_ref[...] = eabs
    es_ref[...] = jnp.sum(eabs, axis=1, keepdims=True)


def _mlp_masses(x, W1, b1, W2, b2, W3, b3):
    nblk = _N // _BN
    out_shapes = (
        jax.ShapeDtypeStruct((_N, _K), jnp.float32),
        jax.ShapeDtypeStruct((_N, 1), jnp.float32),
    )
    return pl.pallas_call(
        _mlp_body,
        grid=(nblk,),
        in_specs=[
            pl.BlockSpec((_BN, _D), lambda i: (i, 0)),
            pl.BlockSpec((_D, _L), lambda i: (0, 0)),
            pl.BlockSpec((1, _L), lambda i: (0, 0)),
            pl.BlockSpec((_L, _L), lambda i: (0, 0)),
            pl.BlockSpec((1, _L), lambda i: (0, 0)),
            pl.BlockSpec((_L, _K), lambda i: (0, 0)),
            pl.BlockSpec((1, _K), lambda i: (0, 0)),
        ],
        out_specs=(
            pl.BlockSpec((_BN, _K), lambda i: (i, 0)),
            pl.BlockSpec((_BN, 1), lambda i: (i, 0)),
        ),
        out_shape=out_shapes,
    )(x, W1, b1.reshape(1, _L), W2, b2.reshape(1, _L), W3, b3.reshape(1, _K))


_SEG16 = _G // _NS


def _sc_body(es_hbm, sid_hbm, stop_hbm, u1_hbm, u2_hbm, eabs_hbm,
             u_out, stop_out, node_out, spec_out,
             sid_v, es_v, cs_v, stop_v, u1_v, u2_v, rows_v, spc_v, tmp_v,
             st_u, st_stop, st_node, st_spec, sem, sem2):
    cidx = lax.axis_index("c")
    wid = lax.axis_index("s")
    iota = lax.iota(jnp.int32, _LANES)

    def splat(ref, g):
        return plsc.load_gather(ref, [jnp.full((_LANES,), g, jnp.int32)])

    h_in = [
        pltpu.async_copy(sid_hbm, sid_v.at[pl.ds(0, _N)], sem2),
        pltpu.async_copy(es_hbm, es_v.at[pl.ds(0, _N)], sem2),
        pltpu.async_copy(stop_hbm, stop_v, sem2),
        pltpu.async_copy(u1_hbm, u1_v, sem2),
        pltpu.async_copy(u2_hbm, u2_v, sem2),
    ]
    for h in h_in:
        h.wait()
    es_v[pl.ds(_N, _LANES)] = jnp.zeros((_LANES,), jnp.float32)

    g0 = wid * _SEG16

    gl = g0 + jnp.minimum(iota, _SEG16)
    pos = jnp.zeros((_LANES,), jnp.int32)
    b = _N // 2
    while b >= 1:
        probe = pos + (b - 1)
        v = plsc.load_gather(sid_v, [probe])
        pos = jnp.where(v < gl, pos + b, pos)
        b //= 2
    pos = jnp.where(gl >= jnp.int32(_G), jnp.int32(_N), pos)
    starts = [pos[jj] for jj in range(_SEG16 + 1)]

    res_u = jnp.zeros((_LANES,), jnp.float32)
    res_stop = jnp.zeros((_LANES,), jnp.float32)
    r1v = jnp.zeros((_LANES,), jnp.float32)

    for j in range(_SEG16):
        s = starts[j]
        e = starts[j + 1]
        nch = lax.div(e - s + (_LANES - 1), _LANES)

        def body_b(k, carryv, s=s, e=e):
            off = s + k * _LANES
            ve = es_v[pl.ds(off, _LANES)]
            mask = (off + iota) < e
            cum = plsc.cumsum(jnp.where(mask, ve, 0.0)) + carryv
            cs_v[pl.ds(off, _LANES)] = cum
            return splat(cs_v, off + (_LANES - 1))

        tvec = pl.loop(0, nch,
                       init_carry=jnp.zeros((_LANES,), jnp.float32))(body_b)

        expstop = jnp.exp(splat(stop_v, g0 + j))
        norm_vec = tvec + expstop
        res_u = jnp.where(iota == j, jnp.float32(1.0) / norm_vec, res_u)
        res_stop = jnp.where(iota == j, expstop / norm_vec, res_stop)
        r1j = tvec * (jnp.float32(1.0) - splat(u1_v, g0 + j))
        r1v = jnp.where(iota == j, r1j, r1v)

    s_vec = pos
    e_vec = jnp.zeros((_LANES,), jnp.int32)
    for j in range(_SEG16):
        e_vec = jnp.where(iota == j, starts[j + 1], e_vec)
    span = e_vec - s_vec
    o = jnp.zeros((_LANES,), jnp.int32)
    b = _N // 2
    while b >= 1:
        cand = o + b
        probe = jnp.minimum(s_vec + cand - 1, jnp.int32(_N - 1))
        v = plsc.load_gather(cs_v, [probe])
        ok = jnp.logical_and(cand <= span, v < r1v)
        o = jnp.where(ok, cand, o)
        b //= 2
    node_vec = s_vec + jnp.minimum(o, span - 1)
    res_node = jnp.where(iota < _SEG16, node_vec, 0)

    handles = [
        pltpu.async_copy(eabs_hbm.at[node_vec[j]], rows_v.at[j], sem)
        for j in range(_SEG16)
    ]
    for h in handles:
        h.wait()

    res_spec = jnp.zeros((_LANES,), jnp.int32)
    for j in range(_SEG16):
        carryv = jnp.zeros((_LANES,), jnp.float32)
        for kk in range(_K // _LANES):
            cum = plsc.cumsum(rows_v[j, pl.ds(kk * _LANES, _LANES)]) + carryv
            spc_v[pl.ds(kk * _LANES, _LANES)] = cum
            carryv = splat(spc_v, kk * _LANES + (_LANES - 1))
        r2v = carryv * (jnp.float32(1.0) - splat(u2_v, g0 + j))
        cnt = jnp.zeros((_LANES,), jnp.int32)
        for kk in range(_K // _LANES):
            cnt = cnt + (spc_v[pl.ds(kk * _LANES, _LANES)] < r2v).astype(jnp.int32)
        tmp_v[...] = plsc.cumsum(cnt)
        spv = jnp.minimum(splat(tmp_v, _LANES - 1), jnp.int32(_K - 1))
        res_spec = jnp.where(iota == j, spv, res_spec)

    @pl.when(cidx == 0)
    def _publish():
        st_u[...] = res_u
        st_stop[...] = res_stop
        st_node[...] = res_node
        st_spec[...] = res_spec
        hs = [
            pltpu.async_copy(st_u.at[pl.ds(0, _SEG16)],
                             u_out.at[pl.ds(g0, _SEG16)], sem),
            pltpu.async_copy(st_stop.at[pl.ds(0, _SEG16)],
                             stop_out.at[pl.ds(g0, _SEG16)], sem),
            pltpu.async_copy(st_node.at[pl.ds(0, _SEG16)],
                             node_out.at[pl.ds(g0, _SEG16)], sem),
            pltpu.async_copy(st_spec.at[pl.ds(0, _SEG16)],
                             spec_out.at[pl.ds(g0, _SEG16)], sem),
        ]
        for h in hs:
            h.wait()


def _sc_segment_sample(es, sid, stop, u1, u2, eabs):
    mesh = plsc.VectorSubcoreMesh(core_axis_name="c", subcore_axis_name="s")
    fn = pl.kernel(
        _sc_body,
        out_type=[
            jax.ShapeDtypeStruct((_G,), jnp.float32),
            jax.ShapeDtypeStruct((_G,), jnp.float32),
            jax.ShapeDtypeStruct((_G,), jnp.int32),
            jax.ShapeDtypeStruct((_G,), jnp.int32),
        ],
        mesh=mesh,
        compiler_params=pltpu.CompilerParams(needs_layout_passes=False),
        scratch_types=[
            pltpu.VMEM((_N + _LANES,), jnp.int32),
            pltpu.VMEM((_N + _LANES,), jnp.float32),
            pltpu.VMEM((_N + _LANES,), jnp.float32),
            pltpu.VMEM((_G,), jnp.float32),
            pltpu.VMEM((_G,), jnp.float32),
            pltpu.VMEM((_G,), jnp.float32),
            pltpu.VMEM((_SEG16, _K), jnp.float32),
            pltpu.VMEM((_K,), jnp.float32),
            pltpu.VMEM((_LANES,), jnp.int32),
            pltpu.VMEM((_LANES,), jnp.float32),
            pltpu.VMEM((_LANES,), jnp.float32),
            pltpu.VMEM((_LANES,), jnp.int32),
            pltpu.VMEM((_LANES,), jnp.int32),
            pltpu.SemaphoreType.DMA,
            pltpu.SemaphoreType.DMA,
        ],
    )
    return fn(es, sid, stop, u1, u2, eabs)


def _probs_body(e_ref, sid_ref, u_ref, out_ref):
    sid = sid_ref[...]
    g = lax.broadcasted_iota(jnp.int32, (1, _G), 1)
    onehot = (sid == g).astype(jnp.float32)
    ucol = jnp.transpose(u_ref[...], (1, 0))
    t = jnp.dot(onehot, ucol, preferred_element_type=jnp.float32)
    out_ref[...] = e_ref[...] * t


_BN2 = 4096


def _probs(eabs, sid2d, U):
    return pl.pallas_call(
        _probs_body,
        grid=(_N // _BN2,),
        in_specs=[
            pl.BlockSpec((_BN2, _K), lambda i: (i, 0)),
            pl.BlockSpec((_BN2, 1), lambda i: (i, 0)),
            pl.BlockSpec((1, _G), lambda i: (0, 0)),
        ],
        out_specs=pl.BlockSpec((_BN2, _K), lambda i: (i, 0)),
        out_shape=jax.ShapeDtypeStruct((_N, _K), jnp.float32),
    )(eabs, sid2d, U.reshape(1, _G))


def kernel(node_embeddings, stop_logits, segment_ids, W1, b1, W2, b2, W3, b3):
    eabs, es2 = _mlp_masses(node_embeddings, W1, b1, W2, b2, W3, b3)
    U, stop_probs, node_indices, species_indices = _sc_segment_sample(
        es2.reshape(_N), segment_ids, stop_logits,
        jnp.asarray(_U1), jnp.asarray(_U2), eabs)
    species_probs = _probs(eabs, segment_ids.reshape(_N, 1), U)
    return species_probs, stop_probs, node_indices, species_indices
